# baseline (speedup 1.0000x reference)
import jax
import jax.numpy as jnp
from jax.experimental import pallas as pl
from jax.experimental.pallas import tpu as pltpu

_ROWS = 1024
_LBL = 512
_SCH = 8
_W = _ROWS // _SCH


def _loss_body(p_hbm, s_hbm, a_hbm, out_ref, p_v, a_v, s_v, cand_v, acc_v, sems):
    p_copies = [
        pltpu.make_async_copy(
            p_hbm.at[pl.ds(h * (_ROWS // 2), _ROWS // 2), :],
            p_v.at[pl.ds(h * (_ROWS // 2), _ROWS // 2), :],
            sems.at[_SCH + 2 + h],
        )
        for h in range(2)
    ]
    a_copy = pltpu.make_async_copy(a_hbm, a_v, sems.at[_SCH + 1])
    s_copies = [
        pltpu.make_async_copy(
            s_hbm.at[:, pl.ds(k * _W, _W)],
            s_v.at[:, pl.ds(k * _W, _W)],
            sems.at[k],
        )
        for k in range(_SCH)
    ]
    for c in p_copies:
        c.start()
    a_copy.start()
    for c in s_copies:
        c.start()

    for c in p_copies:
        c.wait()
    a_copy.wait()
    inv_l = jnp.float32(1.0 / _LBL)
    cand_v[...] = (
        jnp.dot(p_v[...], a_v[...], preferred_element_type=jnp.float32) * inv_l
    )

    rs = None
    for k in range(_SCH):
        s_copies[k].wait()
        s_blk = s_v[:, pl.ds(k * _W, _W)]
        part = jnp.dot(
            s_blk,
            cand_v[pl.ds(k * _W, _W), :],
            preferred_element_type=jnp.float32,
        )
        rs_part = jnp.sum(s_blk, axis=1)
        if k == 0:
            acc_v[...] = part
            rs = rs_part
        else:
            acc_v[...] += part
            rs = rs + rs_part

    diff = p_v[...] - acc_v[...]
    sq = jnp.sum(diff * diff, axis=1)
    norms = jnp.sqrt(sq)
    mask = rs != 0
    cnt = jnp.sum(mask.astype(jnp.float32))
    total = jnp.sum(jnp.where(mask, norms, jnp.float32(0.0)))
    out_ref[...] = jnp.reshape(total / cnt, (1, 1))


def kernel(predicts, similarities, adjList):
    out = pl.pallas_call(
        _loss_body,
        in_specs=[
            pl.BlockSpec(memory_space=pltpu.MemorySpace.HBM),
            pl.BlockSpec(memory_space=pltpu.MemorySpace.HBM),
            pl.BlockSpec(memory_space=pltpu.MemorySpace.HBM),
        ],
        out_specs=pl.BlockSpec(memory_space=pltpu.VMEM),
        out_shape=jax.ShapeDtypeStruct((1, 1), jnp.float32),
        scratch_shapes=[
            pltpu.VMEM((_ROWS, _LBL), jnp.float32),
            pltpu.VMEM((_LBL, _LBL), jnp.float32),
            pltpu.VMEM((_ROWS, _ROWS), jnp.float32),
            pltpu.VMEM((_ROWS, _LBL), jnp.float32),
            pltpu.VMEM((_ROWS, _LBL), jnp.float32),
            pltpu.SemaphoreType.DMA((4 + _SCH,)),
        ],
    )(predicts, similarities, adjList)
    return out[0, 0]


# speedup vs baseline: 1.2645x; 1.2645x over previous
import jax
import jax.numpy as jnp
from jax.experimental import pallas as pl
from jax.experimental.pallas import tpu as pltpu

_ROWS = 1024
_LBL = 512
_SCH = 2
_W = _ROWS // _SCH


def _loss_body(p_hbm, s_hbm, a_hbm, out_ref, p_v, a_v, s_v, cand_v, acc_v, sems):
    p_copy = pltpu.make_async_copy(p_hbm, p_v, sems.at[_SCH])
    a_copy = pltpu.make_async_copy(a_hbm, a_v, sems.at[_SCH + 1])
    s_copies = [
        pltpu.make_async_copy(
            s_hbm.at[:, pl.ds(k * _W, _W)],
            s_v.at[:, pl.ds(k * _W, _W)],
            sems.at[k],
        )
        for k in range(_SCH)
    ]
    p_copy.start()
    a_copy.start()
    for c in s_copies:
        c.start()

    p_copy.wait()
    a_copy.wait()
    inv_l = jnp.float32(1.0 / _LBL)
    cand_v[...] = (
        jnp.dot(p_v[...], a_v[...], preferred_element_type=jnp.float32) * inv_l
    )

    rs = None
    for k in range(_SCH):
        s_copies[k].wait()
        s_blk = s_v[:, pl.ds(k * _W, _W)]
        part = jnp.dot(
            s_blk,
            cand_v[pl.ds(k * _W, _W), :],
            preferred_element_type=jnp.float32,
        )
        rs_part = jnp.sum(s_blk, axis=1)
        if k == 0:
            acc_v[...] = part
            rs = rs_part
        else:
            acc_v[...] += part
            rs = rs + rs_part

    diff = p_v[...] - acc_v[...]
    sq = jnp.sum(diff * diff, axis=1)
    norms = jnp.sqrt(sq)
    mask = rs != 0
    cnt = jnp.sum(mask.astype(jnp.float32))
    total = jnp.sum(jnp.where(mask, norms, jnp.float32(0.0)))
    out_ref[...] = jnp.reshape(total / cnt, (1, 1))


def kernel(predicts, similarities, adjList):
    out = pl.pallas_call(
        _loss_body,
        in_specs=[
            pl.BlockSpec(memory_space=pltpu.MemorySpace.HBM),
            pl.BlockSpec(memory_space=pltpu.MemorySpace.HBM),
            pl.BlockSpec(memory_space=pltpu.MemorySpace.HBM),
        ],
        out_specs=pl.BlockSpec(memory_space=pltpu.VMEM),
        out_shape=jax.ShapeDtypeStruct((1, 1), jnp.float32),
        scratch_shapes=[
            pltpu.VMEM((_ROWS, _LBL), jnp.float32),
            pltpu.VMEM((_LBL, _LBL), jnp.float32),
            pltpu.VMEM((_ROWS, _ROWS), jnp.float32),
            pltpu.VMEM((_ROWS, _LBL), jnp.float32),
            pltpu.VMEM((_ROWS, _LBL), jnp.float32),
            pltpu.SemaphoreType.DMA((2 + _SCH,)),
        ],
    )(predicts, similarities, adjList)
    return out[0, 0]
